# baseline (speedup 1.0000x reference)
import functools

import jax
import jax.numpy as jnp
from jax import lax
from jax.experimental import pallas as pl
from jax.experimental.pallas import tpu as pltpu
from jax.experimental.pallas import tpu_sc as plsc

N_NODES = 2048
D_FEAT = 128
K_SAMPLES = 16
BLK = 256

_NC = 2
_NS = 16
_ROWS_PER_W = N_NODES // (_NC * _NS)
_CHUNK = 8


def _sc_mask_kernel(cs_hbm, zeros_hbm, m_hbm, cs_v, buf0, buf1, sem0, sem1):
    wid = lax.axis_index("s") * _NC + lax.axis_index("c")
    base = wid * _ROWS_PER_W
    pltpu.sync_copy(cs_hbm.at[pl.ds(base * K_SAMPLES, _ROWS_PER_W * K_SAMPLES)],
                    cs_v)
    pltpu.sync_copy(zeros_hbm, buf0)
    pltpu.sync_copy(zeros_hbm, buf1)
    one = jnp.full((K_SAMPLES,), 1.0, jnp.float32)
    zero = jnp.zeros((K_SAMPLES,), jnp.float32)
    bufs, sems = (buf0, buf1), (sem0, sem1)
    n_chunks = _ROWS_PER_W // _CHUNK
    copies = [None, None]
    for c in range(n_chunks):
        b = c % 2
        buf = bufs[b]
        if copies[b] is not None:
            copies[b].wait()
            for r in range(_CHUNK):
                cols = cs_v[pl.ds(((c - 2) * _CHUNK + r) * K_SAMPLES,
                                  K_SAMPLES)]
                plsc.store_scatter(buf, [cols + r * N_NODES], zero)
        for r in range(_CHUNK):
            cols = cs_v[pl.ds((c * _CHUNK + r) * K_SAMPLES, K_SAMPLES)]
            plsc.store_scatter(buf, [cols + r * N_NODES], one)
        copies[b] = pltpu.async_copy(
            buf,
            m_hbm.at[pl.ds((base + c * _CHUNK) * N_NODES, _CHUNK * N_NODES)],
            sems[b])
    copies[(n_chunks - 2) % 2].wait()
    copies[(n_chunks - 1) % 2].wait()


def _select_kernel(cs_ref, x_ref, xt_ref, adj_ref, m_ref, out_ref, mbf_ref):
    @pl.when(pl.program_id(0) == 0)
    def _cvt():
        mbf_ref[...] = m_ref[...].astype(jnp.bfloat16)

    cs = cs_ref[...]
    xb = x_ref[...]
    xt = xt_ref[...]
    adj = adj_ref[...]

    iota = jax.lax.broadcasted_iota(jnp.int32, (BLK, N_NODES), 1)

    sims = jnp.dot(xb, xt, preferred_element_type=jnp.float32)
    rowids = (pl.program_id(0) * BLK
              + jax.lax.broadcasted_iota(jnp.int32, (BLK, N_NODES), 0))
    adj_self = adj + jnp.where(iota == rowids, 1.0, 0.0)
    cnt = jnp.dot(adj_self.astype(jnp.bfloat16), mbf_ref[...],
                  preferred_element_type=jnp.float32)
    present = cnt > 0.0

    neg = jnp.float32(-jnp.inf)
    masked = jnp.where(present, sims, neg)

    top_cols = []
    for _ in range(K_SAMPLES):
        idx = jnp.argmax(masked, axis=1).astype(jnp.int32).reshape(BLK, 1)
        top_cols.append(idx)
        masked = jnp.where(iota == idx, neg, masked)
    topk_idx = jnp.concatenate(top_cols, axis=1)

    pool = jnp.sum(present.astype(jnp.int32), axis=1, keepdims=True)
    has_nb = jnp.max(adj, axis=1, keepdims=True) > 0.0

    out_ref[...] = jnp.where(has_nb, topk_idx, cs)

    @pl.when(jnp.any(pool < K_SAMPLES))
    def _fallback():
        cur = jnp.where(present, iota, N_NODES)
        small_cols = []
        for _ in range(K_SAMPLES):
            mn = jnp.min(cur, axis=1, keepdims=True)
            small_cols.append(mn)
            cur = jnp.where(cur == mn, N_NODES, cur)
        sorted_vals = jnp.concatenate(small_cols, axis=1)

        jr = jax.lax.broadcasted_iota(jnp.int32, (BLK, K_SAMPLES), 1)
        pad_idx = jnp.clip(jr - pool, 0, K_SAMPLES - 1)
        gath = jnp.where(pad_idx == 0, cs[:, 0:1], 0)
        for kk in range(1, K_SAMPLES):
            gath = gath + jnp.where(pad_idx == kk, cs[:, kk : kk + 1], 0)
        padded = jnp.where(jr < pool, sorted_vals, gath)

        selected = jnp.where(pool >= K_SAMPLES, topk_idx, padded)
        out_ref[...] = jnp.where(has_nb, selected, cs)


def kernel(x, adj, current_samples):
    out_dtype = current_samples.dtype
    cs = current_samples.astype(jnp.int32)

    sc_mask = functools.partial(
        pl.kernel,
        mesh=plsc.VectorSubcoreMesh(core_axis_name="c", subcore_axis_name="s"),
        compiler_params=pltpu.CompilerParams(needs_layout_passes=False),
        out_type=jax.ShapeDtypeStruct((N_NODES * N_NODES,), jnp.float32),
        scratch_types=[
            pltpu.VMEM((_ROWS_PER_W * K_SAMPLES,), jnp.int32),
            pltpu.VMEM((_CHUNK * N_NODES,), jnp.float32),
            pltpu.VMEM((_CHUNK * N_NODES,), jnp.float32),
            pltpu.SemaphoreType.DMA,
            pltpu.SemaphoreType.DMA,
        ],
    )(_sc_mask_kernel)
    m = sc_mask(cs.reshape(-1), jnp.zeros((_CHUNK * N_NODES,), jnp.float32))
    m = m.reshape(N_NODES, N_NODES)

    out = pl.pallas_call(
        _select_kernel,
        grid=(N_NODES // BLK,),
        in_specs=[
            pl.BlockSpec((BLK, K_SAMPLES), lambda i: (i, 0)),
            pl.BlockSpec((BLK, D_FEAT), lambda i: (i, 0)),
            pl.BlockSpec((D_FEAT, N_NODES), lambda i: (0, 0)),
            pl.BlockSpec((BLK, N_NODES), lambda i: (i, 0)),
            pl.BlockSpec((N_NODES, N_NODES), lambda i: (0, 0)),
        ],
        out_specs=pl.BlockSpec((BLK, K_SAMPLES), lambda i: (i, 0)),
        out_shape=jax.ShapeDtypeStruct((N_NODES, K_SAMPLES), jnp.int32),
        scratch_shapes=[pltpu.VMEM((N_NODES, N_NODES), jnp.bfloat16)],
    )(cs, x, x.T, adj, m)

    return out.astype(out_dtype)


# speedup vs baseline: 1.0285x; 1.0285x over previous
import functools

import jax
import jax.numpy as jnp
from jax import lax
from jax.experimental import pallas as pl
from jax.experimental.pallas import tpu as pltpu
from jax.experimental.pallas import tpu_sc as plsc

N_NODES = 2048
D_FEAT = 128
K_SAMPLES = 16
BLK = 512

_NC = 2
_NS = 16
_ROWS_PER_W = N_NODES // (_NC * _NS)
_CHUNK = 8


def _sc_mask_kernel(cs_hbm, zeros_hbm, m_hbm, cs_v, buf_v):
    wid = lax.axis_index("s") * _NC + lax.axis_index("c")
    base = wid * _ROWS_PER_W
    pltpu.sync_copy(cs_hbm.at[pl.ds(base * K_SAMPLES, _ROWS_PER_W * K_SAMPLES)],
                    cs_v)
    pltpu.sync_copy(zeros_hbm, buf_v)
    one = jnp.full((K_SAMPLES,), 1.0, jnp.float32)
    zero = jnp.zeros((K_SAMPLES,), jnp.float32)
    for c in range(_ROWS_PER_W // _CHUNK):
        for r in range(_CHUNK):
            cols = cs_v[pl.ds((c * _CHUNK + r) * K_SAMPLES, K_SAMPLES)]
            plsc.store_scatter(buf_v, [cols + r * N_NODES], one)
        pltpu.sync_copy(
            buf_v,
            m_hbm.at[pl.ds((base + c * _CHUNK) * N_NODES, _CHUNK * N_NODES)])
        for r in range(_CHUNK):
            cols = cs_v[pl.ds((c * _CHUNK + r) * K_SAMPLES, K_SAMPLES)]
            plsc.store_scatter(buf_v, [cols + r * N_NODES], zero)


def _select_kernel(cs_ref, x_ref, xt_ref, adj_ref, m_ref, out_ref, mbf_ref):
    @pl.when(pl.program_id(0) == 0)
    def _cvt():
        mbf_ref[...] = m_ref[...].astype(jnp.bfloat16)

    cs = cs_ref[...]
    xb = x_ref[...]
    xt = xt_ref[...]
    adj = adj_ref[...]

    iota = jax.lax.broadcasted_iota(jnp.int32, (BLK, N_NODES), 1)

    sims = jnp.dot(xb, xt, preferred_element_type=jnp.float32)
    rowids = (pl.program_id(0) * BLK
              + jax.lax.broadcasted_iota(jnp.int32, (BLK, N_NODES), 0))
    adj_self = adj + jnp.where(iota == rowids, 1.0, 0.0)
    cnt = jnp.dot(adj_self.astype(jnp.bfloat16), mbf_ref[...],
                  preferred_element_type=jnp.float32)
    present = cnt > 0.0

    neg = jnp.float32(-jnp.inf)
    masked = jnp.where(present, sims, neg)

    top_cols = []
    for _ in range(K_SAMPLES):
        idx = jnp.argmax(masked, axis=1).astype(jnp.int32).reshape(BLK, 1)
        top_cols.append(idx)
        masked = jnp.where(iota == idx, neg, masked)
    topk_idx = jnp.concatenate(top_cols, axis=1)

    pool = jnp.sum(present.astype(jnp.int32), axis=1, keepdims=True)
    has_nb = jnp.max(adj, axis=1, keepdims=True) > 0.0

    out_ref[...] = jnp.where(has_nb, topk_idx, cs)

    @pl.when(jnp.any(pool < K_SAMPLES))
    def _fallback():
        cur = jnp.where(present, iota, N_NODES)
        small_cols = []
        for _ in range(K_SAMPLES):
            mn = jnp.min(cur, axis=1, keepdims=True)
            small_cols.append(mn)
            cur = jnp.where(cur == mn, N_NODES, cur)
        sorted_vals = jnp.concatenate(small_cols, axis=1)

        jr = jax.lax.broadcasted_iota(jnp.int32, (BLK, K_SAMPLES), 1)
        pad_idx = jnp.clip(jr - pool, 0, K_SAMPLES - 1)
        gath = jnp.where(pad_idx == 0, cs[:, 0:1], 0)
        for kk in range(1, K_SAMPLES):
            gath = gath + jnp.where(pad_idx == kk, cs[:, kk : kk + 1], 0)
        padded = jnp.where(jr < pool, sorted_vals, gath)

        selected = jnp.where(pool >= K_SAMPLES, topk_idx, padded)
        out_ref[...] = jnp.where(has_nb, selected, cs)


def kernel(x, adj, current_samples):
    out_dtype = current_samples.dtype
    cs = current_samples.astype(jnp.int32)

    sc_mask = functools.partial(
        pl.kernel,
        mesh=plsc.VectorSubcoreMesh(core_axis_name="c", subcore_axis_name="s"),
        compiler_params=pltpu.CompilerParams(needs_layout_passes=False),
        out_type=jax.ShapeDtypeStruct((N_NODES * N_NODES,), jnp.float32),
        scratch_types=[
            pltpu.VMEM((_ROWS_PER_W * K_SAMPLES,), jnp.int32),
            pltpu.VMEM((_CHUNK * N_NODES,), jnp.float32),
        ],
    )(_sc_mask_kernel)
    m = sc_mask(cs.reshape(-1), jnp.zeros((_CHUNK * N_NODES,), jnp.float32))
    m = m.reshape(N_NODES, N_NODES)

    out = pl.pallas_call(
        _select_kernel,
        grid=(N_NODES // BLK,),
        in_specs=[
            pl.BlockSpec((BLK, K_SAMPLES), lambda i: (i, 0)),
            pl.BlockSpec((BLK, D_FEAT), lambda i: (i, 0)),
            pl.BlockSpec((D_FEAT, N_NODES), lambda i: (0, 0)),
            pl.BlockSpec((BLK, N_NODES), lambda i: (i, 0)),
            pl.BlockSpec((N_NODES, N_NODES), lambda i: (0, 0)),
        ],
        out_specs=pl.BlockSpec((BLK, K_SAMPLES), lambda i: (i, 0)),
        out_shape=jax.ShapeDtypeStruct((N_NODES, K_SAMPLES), jnp.int32),
        scratch_shapes=[pltpu.VMEM((N_NODES, N_NODES), jnp.bfloat16)],
    )(cs, x, x.T, adj, m)

    return out.astype(out_dtype)
